# Initial kernel scaffold; baseline (speedup 1.0000x reference)
import jax
import jax.numpy as jnp
from jax.experimental import pallas as pl


def kernel(user_emb, item_emb, edge_vals, edge_index):
    raise NotImplementedError("write your pallas kernel here")


# speedup vs baseline: 7.0167x; 7.0167x over previous
import functools

import jax
import jax.numpy as jnp
from jax import lax
from jax.experimental import pallas as pl
from jax.experimental.pallas import tpu as pltpu
from jax.experimental.pallas import tpu_sc as plsc

_U = 25000
_I = 25000
_N = _U + _I
_E = 1600000
_D = 32
_LAYERS = 3

_NW = 32
_EW = _E // _NW
_B = 128
_NB = -(-_EW // _B)
_EWP = _NB * _B
_NP = 50048
_RPT = _NP // 16
_ZR = 136


def _sc_layer_body(ego, cols, rows, vals, out, acc, colv, rowv, valv, msg,
                   zbuf, sem):
    cid = lax.axis_index("c")
    sid = lax.axis_index("s")
    wid = sid * 2 + cid

    def _zb(i, c):
        zbuf[i, pl.ds(0, 16)] = jnp.zeros((16,), jnp.float32)
        zbuf[i, pl.ds(16, 16)] = jnp.zeros((16,), jnp.float32)
        return c

    lax.fori_loop(0, _ZR, _zb, 0)
    base = sid * _RPT

    def _zc(i, c):
        pltpu.sync_copy(zbuf, acc.at[pl.ds(base + i * _ZR, _ZR)])
        return c

    lax.fori_loop(0, _RPT // _ZR, _zc, 0)
    plsc.subcore_barrier()

    def _batch(j, c):
        pltpu.sync_copy(cols.at[wid, j], colv)
        pltpu.sync_copy(rows.at[wid, j], rowv)
        pltpu.sync_copy(vals.at[wid, j], valv)
        pltpu.async_copy(ego.at[colv], msg, sem).wait()

        def _scale(g, cc):
            vg = valv[pl.ds(g * 16, 16)]
            for e2 in range(16):
                e = g * 16 + e2
                v = vg[e2]
                msg[e, pl.ds(0, 16)] = msg[e, pl.ds(0, 16)] * v
                msg[e, pl.ds(16, 16)] = msg[e, pl.ds(16, 16)] * v
            return cc

        lax.fori_loop(0, _B // 16, _scale, 0)
        pltpu.sync_copy(msg, acc.at[rowv], add=True)
        return c

    lax.fori_loop(0, _NB, _batch, 0)
    plsc.subcore_barrier()
    pltpu.sync_copy(acc.at[pl.ds(base, _RPT)],
                    out.at[cid, pl.ds(base, _RPT)])


def _make_sc_layer():
    mesh = plsc.VectorSubcoreMesh(core_axis_name="c", subcore_axis_name="s")
    return functools.partial(
        pl.kernel,
        mesh=mesh,
        compiler_params=pltpu.CompilerParams(use_tc_tiling_on_sc=False),
        out_type=jax.ShapeDtypeStruct((2, _NP, _D), jnp.float32),
        scratch_types=[
            pltpu.VMEM_SHARED((_NP, _D), jnp.float32),
            pltpu.VMEM((_B,), jnp.int32),
            pltpu.VMEM((_B,), jnp.int32),
            pltpu.VMEM((_B,), jnp.float32),
            pltpu.VMEM((_B, _D), jnp.float32),
            pltpu.VMEM((_ZR, _D), jnp.float32),
            pltpu.SemaphoreType.DMA,
        ],
    )(_sc_layer_body)


def _combine2_body(a_ref, b_ref, o_ref):
    o_ref[...] = a_ref[...] + b_ref[...]


def _combine2(parts):
    rb = 6256
    return pl.pallas_call(
        _combine2_body,
        grid=(_NP // rb,),
        in_specs=[pl.BlockSpec((rb, _D), lambda i: (i, 0)),
                  pl.BlockSpec((rb, _D), lambda i: (i, 0))],
        out_specs=pl.BlockSpec((rb, _D), lambda i: (i, 0)),
        out_shape=jax.ShapeDtypeStruct((_NP, _D), jnp.float32),
    )(parts[0], parts[1])


def _mean_body(e0, e1, e2, p3a, p3b, o_ref):
    o_ref[...] = (e0[...] + e1[...] + e2[...] + p3a[...] + p3b[...]) * 0.25


def _mean4(e0, e1, e2, p3):
    rb = 6256
    spec = pl.BlockSpec((rb, _D), lambda i: (i, 0))
    return pl.pallas_call(
        _mean_body,
        grid=(_NP // rb,),
        in_specs=[spec] * 5,
        out_specs=spec,
        out_shape=jax.ShapeDtypeStruct((_NP, _D), jnp.float32),
    )(e0, e1, e2, p3[0], p3[1])


def kernel(user_emb, item_emb, edge_vals, edge_index):
    ego0 = jnp.concatenate([user_emb, item_emb], axis=0)
    ego0 = jnp.pad(ego0, ((0, _NP - _N), (0, 0)))

    pad = _EWP - _EW
    cols = edge_index[1].reshape(_NW, _EW)
    rows = edge_index[0].reshape(_NW, _EW)
    vals = edge_vals.reshape(_NW, _EW)
    zi = jnp.zeros((_NW, pad), jnp.int32)
    zf = jnp.zeros((_NW, pad), jnp.float32)
    cols = jnp.concatenate([cols, zi], axis=1).reshape(_NW, _NB, _B)
    rows = jnp.concatenate([rows, zi], axis=1).reshape(_NW, _NB, _B)
    vals = jnp.concatenate([vals, zf], axis=1).reshape(_NW, _NB, _B)

    layer = _make_sc_layer()
    egos = [ego0]
    parts = None
    ego = ego0
    for l in range(_LAYERS):
        parts = layer(ego, cols, rows, vals)
        if l < _LAYERS - 1:
            ego = _combine2(parts)
            egos.append(ego)

    all_e = _mean4(egos[0], egos[1], egos[2], parts)
    return (all_e[:_U], all_e[_U:_N])


# speedup vs baseline: 14.6439x; 2.0870x over previous
import functools

import jax
import jax.numpy as jnp
from jax import lax
from jax.experimental import pallas as pl
from jax.experimental.pallas import tpu as pltpu
from jax.experimental.pallas import tpu_sc as plsc

_U = 25000
_I = 25000
_N = _U + _I
_E = 1600000
_D = 32
_LAYERS = 3

_NW = 32
_EW = _E // _NW
_B = 128
_S = 8
_NSC = 49
_NB = _S * _NSC
_EWP = _NB * _B
_NP = 50048
_RPT = _NP // 16
_ZR = 136


def _sc_layer_body(ego, cols, rows, vals, out, acc, colv, rowv, valv,
                   m0, m1, zbuf, sg0, sg1, ss0, ss1):
    cid = lax.axis_index("c")
    sid = lax.axis_index("s")
    wid = sid * 2 + cid

    def _zb(i, c):
        zbuf[i, pl.ds(0, 16)] = jnp.zeros((16,), jnp.float32)
        zbuf[i, pl.ds(16, 16)] = jnp.zeros((16,), jnp.float32)
        return c

    lax.fori_loop(0, _ZR, _zb, 0)
    base = sid * _RPT

    def _zc(i, c):
        pltpu.sync_copy(zbuf, acc.at[pl.ds(base + i * _ZR, _ZR)])
        return c

    lax.fori_loop(0, _RPT // _ZR, _zc, 0)
    plsc.subcore_barrier()

    msgs = (m0, m1)
    gsems = (sg0, sg1)
    ssems = (ss0, ss1)

    def _scale(jj, m):
        def _body(g, cc):
            vg = valv[jj, pl.ds(g * 16, 16)]
            for e2 in range(16):
                e = g * 16 + e2
                v = vg[e2]
                m[e, pl.ds(0, 16)] = m[e, pl.ds(0, 16)] * v
                m[e, pl.ds(16, 16)] = m[e, pl.ds(16, 16)] * v
            return cc

        lax.fori_loop(0, _B // 16, _body, 0)

    def _chunk(j, c):
        pltpu.sync_copy(cols.at[wid, j], colv)
        pltpu.sync_copy(rows.at[wid, j], rowv)
        pltpu.sync_copy(vals.at[wid, j], valv)
        h_g = [None, None]
        h_s = [None, None]
        h_g[0] = pltpu.async_copy(ego.at[colv.at[0]], m0, sg0)
        for jj in range(_S):
            b = jj & 1
            nb = (jj + 1) & 1
            if jj + 1 < _S:
                if h_s[nb] is not None:
                    h_s[nb].wait()
                h_g[nb] = pltpu.async_copy(ego.at[colv.at[jj + 1]],
                                           msgs[nb], gsems[nb])
            h_g[b].wait()
            _scale(jj, msgs[b])
            h_s[b] = pltpu.async_copy(msgs[b], acc.at[rowv.at[jj]],
                                      ssems[b], add=True)
        h_s[0].wait()
        h_s[1].wait()
        return c

    lax.fori_loop(0, _NSC, _chunk, 0)
    plsc.subcore_barrier()
    pltpu.sync_copy(acc.at[pl.ds(base, _RPT)],
                    out.at[cid, pl.ds(base, _RPT)])


def _make_sc_layer():
    mesh = plsc.VectorSubcoreMesh(core_axis_name="c", subcore_axis_name="s")
    return functools.partial(
        pl.kernel,
        mesh=mesh,
        compiler_params=pltpu.CompilerParams(use_tc_tiling_on_sc=False),
        out_type=jax.ShapeDtypeStruct((2, _NP, _D), jnp.float32),
        scratch_types=[
            pltpu.VMEM_SHARED((_NP, _D), jnp.float32),
            pltpu.VMEM((_S, _B), jnp.int32),
            pltpu.VMEM((_S, _B), jnp.int32),
            pltpu.VMEM((_S, _B), jnp.float32),
            pltpu.VMEM((_B, _D), jnp.float32),
            pltpu.VMEM((_B, _D), jnp.float32),
            pltpu.VMEM((_ZR, _D), jnp.float32),
            pltpu.SemaphoreType.DMA,
            pltpu.SemaphoreType.DMA,
            pltpu.SemaphoreType.DMA,
            pltpu.SemaphoreType.DMA,
        ],
    )(_sc_layer_body)


def _combine2_body(a_ref, b_ref, o_ref):
    o_ref[...] = a_ref[...] + b_ref[...]


def _combine2(parts):
    rb = 6256
    return pl.pallas_call(
        _combine2_body,
        grid=(_NP // rb,),
        in_specs=[pl.BlockSpec((rb, _D), lambda i: (i, 0)),
                  pl.BlockSpec((rb, _D), lambda i: (i, 0))],
        out_specs=pl.BlockSpec((rb, _D), lambda i: (i, 0)),
        out_shape=jax.ShapeDtypeStruct((_NP, _D), jnp.float32),
    )(parts[0], parts[1])


def _mean_body(e0, e1, e2, p3a, p3b, o_ref):
    o_ref[...] = (e0[...] + e1[...] + e2[...] + p3a[...] + p3b[...]) * 0.25


def _mean4(e0, e1, e2, p3):
    rb = 6256
    spec = pl.BlockSpec((rb, _D), lambda i: (i, 0))
    return pl.pallas_call(
        _mean_body,
        grid=(_NP // rb,),
        in_specs=[spec] * 5,
        out_specs=spec,
        out_shape=jax.ShapeDtypeStruct((_NP, _D), jnp.float32),
    )(e0, e1, e2, p3[0], p3[1])


def kernel(user_emb, item_emb, edge_vals, edge_index):
    ego0 = jnp.concatenate([user_emb, item_emb], axis=0)
    ego0 = jnp.pad(ego0, ((0, _NP - _N), (0, 0)))

    pad = _EWP - _EW
    cols = edge_index[1].reshape(_NW, _EW)
    rows = edge_index[0].reshape(_NW, _EW)
    vals = edge_vals.reshape(_NW, _EW)
    zi = jnp.zeros((_NW, pad), jnp.int32)
    zf = jnp.zeros((_NW, pad), jnp.float32)
    cols = jnp.concatenate([cols, zi], axis=1).reshape(_NW, _NSC, _S, _B)
    rows = jnp.concatenate([rows, zi], axis=1).reshape(_NW, _NSC, _S, _B)
    vals = jnp.concatenate([vals, zf], axis=1).reshape(_NW, _NSC, _S, _B)

    layer = _make_sc_layer()
    egos = [ego0]
    parts = None
    ego = ego0
    for l in range(_LAYERS):
        parts = layer(ego, cols, rows, vals)
        if l < _LAYERS - 1:
            ego = _combine2(parts)
            egos.append(ego)

    all_e = _mean4(egos[0], egos[1], egos[2], parts)
    return (all_e[:_U], all_e[_U:_N])
